# baseline (speedup 1.0000x reference)
import functools

import jax
import jax.numpy as jnp
from jax import lax
from jax.experimental import pallas as pl
from jax.experimental.pallas import tpu as pltpu
from jax.experimental.pallas import tpu_sc as plsc

VOCAB = 100000
NF = 26
ED = 106
B = 16384
H = 10
HP = 16

NC = 2
NS = 16
L = 16
NW = NC * NS

BW = B // NW
C = 128
CPF = BW // C
VB = 12800

PHASES = (8, 8, 10)


def _tc_project(tt, W1p, fa, nf):
    def proj_kernel(w_ref, t_ref, p_ref):
        t = t_ref[0]
        w = w_ref[0]
        p_ref[0] = lax.dot_general(t, w, (((0,), (0,)), ((), ())),
                                   preferred_element_type=jnp.float32)

    nv = (VOCAB + VB - 1) // VB
    return pl.pallas_call(
        proj_kernel,
        grid=(nf, nv),
        in_specs=[
            pl.BlockSpec((1, ED, HP), lambda f, j: (fa + f, 0, 0)),
            pl.BlockSpec((1, ED, VB), lambda f, j: (fa + f, 0, j)),
        ],
        out_specs=pl.BlockSpec((1, VB, HP), lambda f, j: (f, j, 0)),
        out_shape=jax.ShapeDtypeStruct((nf, VOCAB, HP), jnp.float32),
    )(W1p, tt)


def _sc_gather(idxT, P, fa, nf):
    mesh = plsc.VectorSubcoreMesh(core_axis_name="c", subcore_axis_name="s")
    nt = nf * CPF

    @functools.partial(
        pl.kernel,
        out_type=jax.ShapeDtypeStruct((nf * B, HP), jnp.float32),
        mesh=mesh,
        scratch_types=[
            pltpu.VMEM((nf, BW), jnp.int32),
            pltpu.VMEM((C, HP), jnp.float32),
            pltpu.VMEM((C, HP), jnp.float32),
            pltpu.SemaphoreType.DMA,
            pltpu.SemaphoreType.DMA,
        ],
    )
    def gather_kernel(idx_hbm, p_hbm, out_hbm, idx_v, buf0, buf1, gsem, wsem):
        wid = lax.axis_index("c") * NS + lax.axis_index("s")
        b0 = wid * BW

        pltpu.sync_copy(idx_hbm.at[pl.ds(fa, nf), pl.ds(b0, BW)], idx_v)

        bufs = (buf0, buf1)

        def g_start(t, buf):
            f = t // CPF
            c = lax.rem(t, CPF)
            for g in range(C // L):
                xv = idx_v[f, pl.ds(c * C + g * L, L)] + f * VOCAB
                for k in range(L):
                    pltpu.async_copy(
                        p_hbm.at[pl.ds(xv[k], 1)],
                        buf.at[pl.ds(g * L + k, 1)],
                        gsem)

        def g_wait(buf):
            pltpu.make_async_copy(p_hbm.at[pl.ds(0, C)], buf, gsem).wait()

        def w_row0(t):
            f = t // CPF
            c = lax.rem(t, CPF)
            return f * B + b0 + c * C

        def w_start(t, buf):
            pltpu.async_copy(buf, out_hbm.at[pl.ds(w_row0(t), C)], wsem)

        def w_wait(t, buf):
            pltpu.make_async_copy(
                buf, out_hbm.at[pl.ds(w_row0(t), C)], wsem).wait()

        g_start(0, buf0)
        g_start(1, buf1)

        def loop_body(i, _):
            for k in range(2):
                t = 2 * i + k
                buf = bufs[k]
                g_wait(buf)
                w_start(t, buf)
                w_wait(t, buf)

                @pl.when(t + 2 < nt)
                def _():
                    g_start(t + 2, buf)
            return 0
        lax.fori_loop(0, nt // 2, loop_body, 0)

    return gather_kernel(idxT, P.reshape(nf * VOCAB, HP))


def _tc_mlp(gs, b1p, W2p, b2p, W3p, b3):
    BM = 1024

    def mlp_kernel(*refs):
        g_refs = refs[:len(PHASES)]
        b1_ref, w2_ref, b2_ref, w3_ref, b3_ref, o_ref = refs[len(PHASES):]
        acc = None
        for g_ref in g_refs:
            for f in range(g_ref.shape[0]):
                x = g_ref[f]
                acc = x if acc is None else acc + x
        d1 = jnp.maximum(acc + b1_ref[...][None, :], 0.0)
        h2 = lax.dot_general(d1, w2_ref[...], (((1,), (1,)), ((), ())),
                             preferred_element_type=jnp.float32)
        h2 = jnp.maximum(h2 + b2_ref[...][None, :], 0.0)
        o = lax.dot_general(h2, w3_ref[...], (((1,), (1,)), ((), ())),
                            preferred_element_type=jnp.float32)
        o_ref[...] = jax.nn.sigmoid(o[:, 0] + b3_ref[0])

    g_specs = [
        pl.BlockSpec((nf, BM, HP), lambda i: (0, i, 0)) for nf in PHASES
    ]
    return pl.pallas_call(
        mlp_kernel,
        grid=(B // BM,),
        in_specs=g_specs + [
            pl.BlockSpec((HP,), lambda i: (0,)),
            pl.BlockSpec((HP, HP), lambda i: (0, 0)),
            pl.BlockSpec((HP,), lambda i: (0,)),
            pl.BlockSpec((1, HP), lambda i: (0, 0)),
            pl.BlockSpec((1,), lambda i: (0,)),
        ],
        out_specs=pl.BlockSpec((BM,), lambda i: (i,)),
        out_shape=jax.ShapeDtypeStruct((B,), jnp.float32),
    )(*[g.reshape(nf, B, HP) for g, nf in zip(gs, PHASES)],
      b1p, W2p, b2p, W3p, b3)


def kernel(sparse_feature, tables, W1, b1, W2, b2, W3, b3):
    tt = jnp.transpose(tables, (0, 2, 1))
    idxT = sparse_feature.astype(jnp.int32).T
    W1p = jnp.pad(jnp.transpose(W1.reshape(H, NF, ED), (1, 2, 0)),
                  ((0, 0), (0, 0), (0, HP - H)))
    b1p = jnp.pad(b1, (0, HP - H))
    b2p = jnp.pad(b2, (0, HP - H))
    W2p = jnp.pad(W2, ((0, HP - H), (0, HP - H)))
    W3p = jnp.pad(W3, ((0, 0), (0, HP - H)))

    gs = []
    fa = 0
    for nf in PHASES:
        P = _tc_project(tt, W1p, fa, nf)
        gs.append(_sc_gather(idxT, P, fa, nf))
        fa += nf
    return _tc_mlp(gs, b1p, W2p, b2p, W3p, b3)


# speedup vs baseline: 1.0029x; 1.0029x over previous
import functools

import jax
import jax.numpy as jnp
from jax import lax
from jax.experimental import pallas as pl
from jax.experimental.pallas import tpu as pltpu
from jax.experimental.pallas import tpu_sc as plsc

VOCAB = 100000
NF = 26
ED = 106
B = 16384
H = 10
HP = 16

NC = 2
NS = 16
L = 16
NW = NC * NS

BW = B // NW
C = 128
CPF = BW // C
VB = 12800

PHASES = (16, 10)


def _tc_project(tt, W1p, fa, nf):
    def proj_kernel(w_ref, t_ref, p_ref):
        t = t_ref[0]
        w = w_ref[0]
        p_ref[0] = lax.dot_general(t, w, (((0,), (0,)), ((), ())),
                                   preferred_element_type=jnp.float32)

    nv = (VOCAB + VB - 1) // VB
    return pl.pallas_call(
        proj_kernel,
        grid=(nf, nv),
        in_specs=[
            pl.BlockSpec((1, ED, HP), lambda f, j: (fa + f, 0, 0)),
            pl.BlockSpec((1, ED, VB), lambda f, j: (fa + f, 0, j)),
        ],
        out_specs=pl.BlockSpec((1, VB, HP), lambda f, j: (f, j, 0)),
        out_shape=jax.ShapeDtypeStruct((nf, VOCAB, HP), jnp.float32),
    )(W1p, tt)


def _sc_gather(idxT, P, fa, nf):
    mesh = plsc.VectorSubcoreMesh(core_axis_name="c", subcore_axis_name="s")
    nt = nf * CPF

    @functools.partial(
        pl.kernel,
        out_type=jax.ShapeDtypeStruct((nf * B, HP), jnp.float32),
        mesh=mesh,
        scratch_types=[
            pltpu.VMEM((nf, BW), jnp.int32),
            pltpu.VMEM((C, HP), jnp.float32),
            pltpu.VMEM((C, HP), jnp.float32),
            pltpu.VMEM((C, HP), jnp.float32),
            pltpu.VMEM((C, HP), jnp.float32),
            pltpu.SemaphoreType.DMA,
            pltpu.SemaphoreType.DMA,
        ],
    )
    def gather_kernel(idx_hbm, p_hbm, out_hbm, idx_v,
                      buf0, buf1, buf2, buf3, gsem, wsem):
        wid = lax.axis_index("c") * NS + lax.axis_index("s")
        b0 = wid * BW

        pltpu.sync_copy(idx_hbm.at[pl.ds(fa, nf), pl.ds(b0, BW)], idx_v)

        bufs = (buf0, buf1, buf2, buf3)

        def g_start(t, buf):
            f = t // CPF
            c = lax.rem(t, CPF)
            for g in range(C // L):
                xv = idx_v[f, pl.ds(c * C + g * L, L)] + f * VOCAB
                for k in range(L):
                    pltpu.async_copy(
                        p_hbm.at[pl.ds(xv[k], 1)],
                        buf.at[pl.ds(g * L + k, 1)],
                        gsem)

        def g_wait(buf):
            pltpu.make_async_copy(p_hbm.at[pl.ds(0, C)], buf, gsem).wait()

        def w_row0(t):
            f = t // CPF
            c = lax.rem(t, CPF)
            return f * B + b0 + c * C

        def w_start(t, buf):
            pltpu.async_copy(buf, out_hbm.at[pl.ds(w_row0(t), C)], wsem)

        def w_wait(t, buf):
            pltpu.make_async_copy(
                buf, out_hbm.at[pl.ds(w_row0(t), C)], wsem).wait()

        g_start(0, buf0)
        g_start(1, buf1)

        def loop_body(i, _):
            for k in range(4):
                t = 4 * i + k
                buf = bufs[k]
                g_wait(buf)

                @pl.when(t >= 2)
                def _():
                    w_wait(t - 2, bufs[(k + 2) % 4])

                @pl.when(t + 2 < nt)
                def _():
                    g_start(t + 2, bufs[(k + 2) % 4])
                w_start(t, buf)
            return 0
        lax.fori_loop(0, nt // 4, loop_body, 0)
        w_wait(nt - 2, bufs[(nt - 2) % 4])
        w_wait(nt - 1, bufs[(nt - 1) % 4])

    return gather_kernel(idxT, P.reshape(nf * VOCAB, HP))


def _tc_mlp(gs, b1p, W2p, b2p, W3p, b3):
    BM = 1024

    def mlp_kernel(*refs):
        g_refs = refs[:len(PHASES)]
        b1_ref, w2_ref, b2_ref, w3_ref, b3_ref, o_ref = refs[len(PHASES):]
        acc = None
        for g_ref in g_refs:
            for f in range(g_ref.shape[0]):
                x = g_ref[f]
                acc = x if acc is None else acc + x
        d1 = jnp.maximum(acc + b1_ref[...][None, :], 0.0)
        h2 = lax.dot_general(d1, w2_ref[...], (((1,), (1,)), ((), ())),
                             preferred_element_type=jnp.float32)
        h2 = jnp.maximum(h2 + b2_ref[...][None, :], 0.0)
        o = lax.dot_general(h2, w3_ref[...], (((1,), (1,)), ((), ())),
                            preferred_element_type=jnp.float32)
        o_ref[...] = jax.nn.sigmoid(o[:, 0] + b3_ref[0])

    g_specs = [
        pl.BlockSpec((nf, BM, HP), lambda i: (0, i, 0)) for nf in PHASES
    ]
    return pl.pallas_call(
        mlp_kernel,
        grid=(B // BM,),
        in_specs=g_specs + [
            pl.BlockSpec((HP,), lambda i: (0,)),
            pl.BlockSpec((HP, HP), lambda i: (0, 0)),
            pl.BlockSpec((HP,), lambda i: (0,)),
            pl.BlockSpec((1, HP), lambda i: (0, 0)),
            pl.BlockSpec((1,), lambda i: (0,)),
        ],
        out_specs=pl.BlockSpec((BM,), lambda i: (i,)),
        out_shape=jax.ShapeDtypeStruct((B,), jnp.float32),
    )(*[g.reshape(nf, B, HP) for g, nf in zip(gs, PHASES)],
      b1p, W2p, b2p, W3p, b3)


def kernel(sparse_feature, tables, W1, b1, W2, b2, W3, b3):
    tt = jnp.transpose(tables, (0, 2, 1))
    idxT = sparse_feature.astype(jnp.int32).T
    W1p = jnp.pad(jnp.transpose(W1.reshape(H, NF, ED), (1, 2, 0)),
                  ((0, 0), (0, 0), (0, HP - H)))
    b1p = jnp.pad(b1, (0, HP - H))
    b2p = jnp.pad(b2, (0, HP - H))
    W2p = jnp.pad(W2, ((0, HP - H), (0, HP - H)))
    W3p = jnp.pad(W3, ((0, 0), (0, HP - H)))

    gs = []
    fa = 0
    for nf in PHASES:
        P = _tc_project(tt, W1p, fa, nf)
        gs.append(_sc_gather(idxT, P, fa, nf))
        fa += nf
    return _tc_mlp(gs, b1p, W2p, b2p, W3p, b3)


# speedup vs baseline: 1.0140x; 1.0111x over previous
import functools

import jax
import jax.numpy as jnp
from jax import lax
from jax.experimental import pallas as pl
from jax.experimental.pallas import tpu as pltpu
from jax.experimental.pallas import tpu_sc as plsc

VOCAB = 100000
NF = 26
ED = 106
B = 16384
H = 10
HP = 16

NC = 2
NS = 16
L = 16
NW = NC * NS

BW = B // NW
C = 128
CPF = BW // C
VB = 25600

PHASES = (16, 10)


def _tc_project(tt, W1p, fa, nf):
    def proj_kernel(w_ref, t_ref, p_ref):
        t = t_ref[0]
        w = w_ref[0]
        p_ref[0] = lax.dot_general(t, w, (((0,), (0,)), ((), ())),
                                   preferred_element_type=jnp.float32)

    nv = (VOCAB + VB - 1) // VB
    return pl.pallas_call(
        proj_kernel,
        grid=(nf, nv),
        in_specs=[
            pl.BlockSpec((1, ED, HP), lambda f, j: (fa + f, 0, 0)),
            pl.BlockSpec((1, ED, VB), lambda f, j: (fa + f, 0, j)),
        ],
        out_specs=pl.BlockSpec((1, VB, HP), lambda f, j: (f, j, 0)),
        out_shape=jax.ShapeDtypeStruct((nf, VOCAB, HP), jnp.float32),
    )(W1p, tt)


def _sc_gather(idxT, P, fa, nf):
    mesh = plsc.VectorSubcoreMesh(core_axis_name="c", subcore_axis_name="s")
    nt = nf * CPF

    @functools.partial(
        pl.kernel,
        out_type=jax.ShapeDtypeStruct((nf * B, HP), jnp.float32),
        mesh=mesh,
        scratch_types=[
            pltpu.VMEM((nf, BW), jnp.int32),
            pltpu.VMEM((C, HP), jnp.float32),
            pltpu.VMEM((C, HP), jnp.float32),
            pltpu.VMEM((C, HP), jnp.float32),
            pltpu.VMEM((C, HP), jnp.float32),
            pltpu.SemaphoreType.DMA,
            pltpu.SemaphoreType.DMA,
        ],
    )
    def gather_kernel(idx_hbm, p_hbm, out_hbm, idx_v,
                      buf0, buf1, buf2, buf3, gsem, wsem):
        wid = lax.axis_index("c") * NS + lax.axis_index("s")
        b0 = wid * BW

        pltpu.sync_copy(idx_hbm.at[pl.ds(fa, nf), pl.ds(b0, BW)], idx_v)

        bufs = (buf0, buf1, buf2, buf3)

        def g_start(t, buf):
            f = t // CPF
            c = lax.rem(t, CPF)
            for g in range(C // L):
                xv = idx_v[f, pl.ds(c * C + g * L, L)] + f * VOCAB
                for k in range(L):
                    pltpu.async_copy(
                        p_hbm.at[pl.ds(xv[k], 1)],
                        buf.at[pl.ds(g * L + k, 1)],
                        gsem)

        def g_wait(buf):
            pltpu.make_async_copy(p_hbm.at[pl.ds(0, C)], buf, gsem).wait()

        def w_row0(t):
            f = t // CPF
            c = lax.rem(t, CPF)
            return f * B + b0 + c * C

        def w_start(t, buf):
            pltpu.async_copy(buf, out_hbm.at[pl.ds(w_row0(t), C)], wsem)

        def w_wait(t, buf):
            pltpu.make_async_copy(
                buf, out_hbm.at[pl.ds(w_row0(t), C)], wsem).wait()

        g_start(0, buf0)
        g_start(1, buf1)

        def loop_body(i, _):
            for k in range(4):
                t = 4 * i + k
                buf = bufs[k]
                g_wait(buf)

                @pl.when(t >= 2)
                def _():
                    w_wait(t - 2, bufs[(k + 2) % 4])

                @pl.when(t + 2 < nt)
                def _():
                    g_start(t + 2, bufs[(k + 2) % 4])
                w_start(t, buf)
            return 0
        lax.fori_loop(0, nt // 4, loop_body, 0)
        w_wait(nt - 2, bufs[(nt - 2) % 4])
        w_wait(nt - 1, bufs[(nt - 1) % 4])

    return gather_kernel(idxT, P.reshape(nf * VOCAB, HP))


def _tc_mlp(gs, b1p, W2p, b2p, W3p, b3):
    BM = 1024

    def mlp_kernel(*refs):
        g_refs = refs[:len(PHASES)]
        b1_ref, w2_ref, b2_ref, w3_ref, b3_ref, o_ref = refs[len(PHASES):]
        acc = None
        for g_ref in g_refs:
            for f in range(g_ref.shape[0]):
                x = g_ref[f]
                acc = x if acc is None else acc + x
        d1 = jnp.maximum(acc + b1_ref[...][None, :], 0.0)
        h2 = lax.dot_general(d1, w2_ref[...], (((1,), (1,)), ((), ())),
                             preferred_element_type=jnp.float32)
        h2 = jnp.maximum(h2 + b2_ref[...][None, :], 0.0)
        o = lax.dot_general(h2, w3_ref[...], (((1,), (1,)), ((), ())),
                            preferred_element_type=jnp.float32)
        o_ref[...] = jax.nn.sigmoid(o[:, 0] + b3_ref[0])

    g_specs = [
        pl.BlockSpec((nf, BM, HP), lambda i: (0, i, 0)) for nf in PHASES
    ]
    return pl.pallas_call(
        mlp_kernel,
        grid=(B // BM,),
        in_specs=g_specs + [
            pl.BlockSpec((HP,), lambda i: (0,)),
            pl.BlockSpec((HP, HP), lambda i: (0, 0)),
            pl.BlockSpec((HP,), lambda i: (0,)),
            pl.BlockSpec((1, HP), lambda i: (0, 0)),
            pl.BlockSpec((1,), lambda i: (0,)),
        ],
        out_specs=pl.BlockSpec((BM,), lambda i: (i,)),
        out_shape=jax.ShapeDtypeStruct((B,), jnp.float32),
    )(*[g.reshape(nf, B, HP) for g, nf in zip(gs, PHASES)],
      b1p, W2p, b2p, W3p, b3)


def kernel(sparse_feature, tables, W1, b1, W2, b2, W3, b3):
    tt = jnp.transpose(tables, (0, 2, 1))
    idxT = sparse_feature.astype(jnp.int32).T
    W1p = jnp.pad(jnp.transpose(W1.reshape(H, NF, ED), (1, 2, 0)),
                  ((0, 0), (0, 0), (0, HP - H)))
    b1p = jnp.pad(b1, (0, HP - H))
    b2p = jnp.pad(b2, (0, HP - H))
    W2p = jnp.pad(W2, ((0, HP - H), (0, HP - H)))
    W3p = jnp.pad(W3, ((0, 0), (0, HP - H)))

    gs = []
    fa = 0
    for nf in PHASES:
        P = _tc_project(tt, W1p, fa, nf)
        gs.append(_sc_gather(idxT, P, fa, nf))
        fa += nf
    return _tc_mlp(gs, b1p, W2p, b2p, W3p, b3)


# speedup vs baseline: 1.0147x; 1.0007x over previous
import functools

import jax
import jax.numpy as jnp
from jax import lax
from jax.experimental import pallas as pl
from jax.experimental.pallas import tpu as pltpu
from jax.experimental.pallas import tpu_sc as plsc

VOCAB = 100000
NF = 26
ED = 106
B = 16384
H = 10
HP = 16

NC = 2
NS = 16
L = 16
NW = NC * NS

BW = B // NW
C = 128
CPF = BW // C
VB = 25600

PHASES = (16, 8, 2)


def _tc_project(tt, W1p, fa, nf):
    def proj_kernel(w_ref, t_ref, p_ref):
        t = t_ref[0]
        w = w_ref[0]
        p_ref[0] = lax.dot_general(t, w, (((0,), (0,)), ((), ())),
                                   preferred_element_type=jnp.float32)

    nv = (VOCAB + VB - 1) // VB
    return pl.pallas_call(
        proj_kernel,
        grid=(nf, nv),
        in_specs=[
            pl.BlockSpec((1, ED, HP), lambda f, j: (fa + f, 0, 0)),
            pl.BlockSpec((1, ED, VB), lambda f, j: (fa + f, 0, j)),
        ],
        out_specs=pl.BlockSpec((1, VB, HP), lambda f, j: (f, j, 0)),
        out_shape=jax.ShapeDtypeStruct((nf, VOCAB, HP), jnp.float32),
    )(W1p, tt)


def _sc_gather(idxT, P, fa, nf):
    mesh = plsc.VectorSubcoreMesh(core_axis_name="c", subcore_axis_name="s")
    nt = nf * CPF

    @functools.partial(
        pl.kernel,
        out_type=jax.ShapeDtypeStruct((nf * B, HP), jnp.float32),
        mesh=mesh,
        scratch_types=[
            pltpu.VMEM((nf, BW), jnp.int32),
            pltpu.VMEM((C, HP), jnp.float32),
            pltpu.VMEM((C, HP), jnp.float32),
            pltpu.VMEM((C, HP), jnp.float32),
            pltpu.VMEM((C, HP), jnp.float32),
            pltpu.SemaphoreType.DMA,
            pltpu.SemaphoreType.DMA,
        ],
    )
    def gather_kernel(idx_hbm, p_hbm, out_hbm, idx_v,
                      buf0, buf1, buf2, buf3, gsem, wsem):
        wid = lax.axis_index("c") * NS + lax.axis_index("s")
        b0 = wid * BW

        pltpu.sync_copy(idx_hbm.at[pl.ds(fa, nf), pl.ds(b0, BW)], idx_v)

        bufs = (buf0, buf1, buf2, buf3)

        def g_start(t, buf):
            f = t // CPF
            c = lax.rem(t, CPF)
            for g in range(C // L):
                xv = idx_v[f, pl.ds(c * C + g * L, L)] + f * VOCAB
                for k in range(L):
                    pltpu.async_copy(
                        p_hbm.at[pl.ds(xv[k], 1)],
                        buf.at[pl.ds(g * L + k, 1)],
                        gsem)

        def g_wait(buf):
            pltpu.make_async_copy(p_hbm.at[pl.ds(0, C)], buf, gsem).wait()

        def w_row0(t):
            f = t // CPF
            c = lax.rem(t, CPF)
            return f * B + b0 + c * C

        def w_start(t, buf):
            pltpu.async_copy(buf, out_hbm.at[pl.ds(w_row0(t), C)], wsem)

        def w_wait(t, buf):
            pltpu.make_async_copy(
                buf, out_hbm.at[pl.ds(w_row0(t), C)], wsem).wait()

        g_start(0, buf0)
        g_start(1, buf1)

        def loop_body(i, _):
            for k in range(4):
                t = 4 * i + k
                buf = bufs[k]
                g_wait(buf)

                @pl.when(t >= 2)
                def _():
                    w_wait(t - 2, bufs[(k + 2) % 4])

                @pl.when(t + 2 < nt)
                def _():
                    g_start(t + 2, bufs[(k + 2) % 4])
                w_start(t, buf)
            return 0
        lax.fori_loop(0, nt // 4, loop_body, 0)
        w_wait(nt - 2, bufs[(nt - 2) % 4])
        w_wait(nt - 1, bufs[(nt - 1) % 4])

    return gather_kernel(idxT, P.reshape(nf * VOCAB, HP))


def _tc_mlp(gs, b1p, W2p, b2p, W3p, b3):
    BM = 1024

    def mlp_kernel(*refs):
        g_refs = refs[:len(PHASES)]
        b1_ref, w2_ref, b2_ref, w3_ref, b3_ref, o_ref = refs[len(PHASES):]
        acc = None
        for g_ref in g_refs:
            for f in range(g_ref.shape[0]):
                x = g_ref[f]
                acc = x if acc is None else acc + x
        d1 = jnp.maximum(acc + b1_ref[...][None, :], 0.0)
        h2 = lax.dot_general(d1, w2_ref[...], (((1,), (1,)), ((), ())),
                             preferred_element_type=jnp.float32)
        h2 = jnp.maximum(h2 + b2_ref[...][None, :], 0.0)
        o = lax.dot_general(h2, w3_ref[...], (((1,), (1,)), ((), ())),
                            preferred_element_type=jnp.float32)
        o_ref[...] = jax.nn.sigmoid(o[:, 0] + b3_ref[0])

    g_specs = [
        pl.BlockSpec((nf, BM, HP), lambda i: (0, i, 0)) for nf in PHASES
    ]
    return pl.pallas_call(
        mlp_kernel,
        grid=(B // BM,),
        in_specs=g_specs + [
            pl.BlockSpec((HP,), lambda i: (0,)),
            pl.BlockSpec((HP, HP), lambda i: (0, 0)),
            pl.BlockSpec((HP,), lambda i: (0,)),
            pl.BlockSpec((1, HP), lambda i: (0, 0)),
            pl.BlockSpec((1,), lambda i: (0,)),
        ],
        out_specs=pl.BlockSpec((BM,), lambda i: (i,)),
        out_shape=jax.ShapeDtypeStruct((B,), jnp.float32),
    )(*[g.reshape(nf, B, HP) for g, nf in zip(gs, PHASES)],
      b1p, W2p, b2p, W3p, b3)


def kernel(sparse_feature, tables, W1, b1, W2, b2, W3, b3):
    tt = jnp.transpose(tables, (0, 2, 1))
    idxT = sparse_feature.astype(jnp.int32).T
    W1p = jnp.pad(jnp.transpose(W1.reshape(H, NF, ED), (1, 2, 0)),
                  ((0, 0), (0, 0), (0, HP - H)))
    b1p = jnp.pad(b1, (0, HP - H))
    b2p = jnp.pad(b2, (0, HP - H))
    W2p = jnp.pad(W2, ((0, HP - H), (0, HP - H)))
    W3p = jnp.pad(W3, ((0, 0), (0, HP - H)))

    gs = []
    fa = 0
    for nf in PHASES:
        P = _tc_project(tt, W1p, fa, nf)
        gs.append(_sc_gather(idxT, P, fa, nf))
        fa += nf
    return _tc_mlp(gs, b1p, W2p, b2p, W3p, b3)


# speedup vs baseline: 1.0149x; 1.0002x over previous
import functools

import jax
import jax.numpy as jnp
from jax import lax
from jax.experimental import pallas as pl
from jax.experimental.pallas import tpu as pltpu
from jax.experimental.pallas import tpu_sc as plsc

VOCAB = 100000
NF = 26
ED = 106
B = 16384
H = 10
HP = 16

NC = 2
NS = 16
L = 16
NW = NC * NS

BW = B // NW
C = 128
CPF = BW // C
VB = 25600

PHASES = (16, 10)


def _tc_project(tt, W1p, fa, nf):
    def proj_kernel(w_ref, t_ref, p_ref):
        t = t_ref[0]
        w = w_ref[0]
        p_ref[0] = lax.dot_general(t, w, (((0,), (0,)), ((), ())),
                                   preferred_element_type=jnp.float32)

    nv = (VOCAB + VB - 1) // VB
    return pl.pallas_call(
        proj_kernel,
        grid=(nf, nv),
        in_specs=[
            pl.BlockSpec((1, ED, HP), lambda f, j: (fa + f, 0, 0)),
            pl.BlockSpec((1, ED, VB), lambda f, j: (fa + f, 0, j)),
        ],
        out_specs=pl.BlockSpec((1, VB, HP), lambda f, j: (f, j, 0)),
        out_shape=jax.ShapeDtypeStruct((nf, VOCAB, HP), jnp.float32),
    )(W1p, tt)


def _sc_gather(idxT, P, fa, nf):
    mesh = plsc.VectorSubcoreMesh(core_axis_name="c", subcore_axis_name="s")
    nt = nf * CPF

    @functools.partial(
        pl.kernel,
        out_type=jax.ShapeDtypeStruct((nf * B, HP), jnp.float32),
        mesh=mesh,
        scratch_types=[
            pltpu.VMEM((nf, BW), jnp.int32),
            pltpu.VMEM((C, HP), jnp.float32),
            pltpu.VMEM((C, HP), jnp.float32),
            pltpu.VMEM((C, HP), jnp.float32),
            pltpu.VMEM((C, HP), jnp.float32),
            pltpu.SemaphoreType.DMA,
            pltpu.SemaphoreType.DMA,
        ],
    )
    def gather_kernel(idx_hbm, p_hbm, out_hbm, idx_v,
                      buf0, buf1, buf2, buf3, gsem, wsem):
        wid = lax.axis_index("c") * NS + lax.axis_index("s")
        b0 = wid * BW

        pltpu.sync_copy(idx_hbm.at[pl.ds(fa, nf), pl.ds(b0, BW)], idx_v)

        bufs = (buf0, buf1, buf2, buf3)

        def g_start(t, buf):
            f = t // CPF
            c = lax.rem(t, CPF)
            for g in range(C // L):
                xv = idx_v[f, pl.ds(c * C + g * L, L)] + f * VOCAB
                for k in range(L):
                    pltpu.async_copy(
                        p_hbm.at[pl.ds(xv[k], 1)],
                        buf.at[pl.ds(g * L + k, 1)],
                        gsem)

        def g_wait(buf):
            pltpu.make_async_copy(p_hbm.at[pl.ds(0, C)], buf, gsem).wait()

        def w_row0(t):
            f = t // CPF
            c = lax.rem(t, CPF)
            return f * B + b0 + c * C

        def w_start(t, buf):
            pltpu.async_copy(buf, out_hbm.at[pl.ds(w_row0(t), C)], wsem)

        def w_wait(t, buf):
            pltpu.make_async_copy(
                buf, out_hbm.at[pl.ds(w_row0(t), C)], wsem).wait()

        g_start(0, buf0)
        g_start(1, buf1)

        def loop_body(i, _):
            for k in range(4):
                t = 4 * i + k
                buf = bufs[k]
                g_wait(buf)

                @pl.when(t >= 2)
                def _():
                    w_wait(t - 2, bufs[(k + 2) % 4])

                @pl.when(t + 2 < nt)
                def _():
                    g_start(t + 2, bufs[(k + 2) % 4])
                w_start(t, buf)
            return 0
        lax.fori_loop(0, nt // 4, loop_body, 0)
        w_wait(nt - 2, bufs[(nt - 2) % 4])
        w_wait(nt - 1, bufs[(nt - 1) % 4])

    return gather_kernel(idxT, P.reshape(nf * VOCAB, HP))


def _tc_mlp(gs, b1p, W2p, b2p, W3p, b3):
    BM = 1024

    def mlp_kernel(*refs):
        g_refs = refs[:len(PHASES)]
        b1_ref, w2_ref, b2_ref, w3_ref, b3_ref, o_ref = refs[len(PHASES):]
        acc = None
        for g_ref in g_refs:
            for f in range(g_ref.shape[0]):
                x = g_ref[f]
                acc = x if acc is None else acc + x
        d1 = jnp.maximum(acc + b1_ref[...][None, :], 0.0)
        h2 = lax.dot_general(d1, w2_ref[...], (((1,), (1,)), ((), ())),
                             preferred_element_type=jnp.float32)
        h2 = jnp.maximum(h2 + b2_ref[...][None, :], 0.0)
        o = lax.dot_general(h2, w3_ref[...], (((1,), (1,)), ((), ())),
                            preferred_element_type=jnp.float32)
        o_ref[...] = jax.nn.sigmoid(o[:, 0] + b3_ref[0])

    g_specs = [
        pl.BlockSpec((nf, BM, HP), lambda i: (0, i, 0)) for nf in PHASES
    ]
    return pl.pallas_call(
        mlp_kernel,
        grid=(B // BM,),
        in_specs=g_specs + [
            pl.BlockSpec((HP,), lambda i: (0,)),
            pl.BlockSpec((HP, HP), lambda i: (0, 0)),
            pl.BlockSpec((HP,), lambda i: (0,)),
            pl.BlockSpec((1, HP), lambda i: (0, 0)),
            pl.BlockSpec((1,), lambda i: (0,)),
        ],
        out_specs=pl.BlockSpec((BM,), lambda i: (i,)),
        out_shape=jax.ShapeDtypeStruct((B,), jnp.float32),
    )(*[g.reshape(nf, B, HP) for g, nf in zip(gs, PHASES)],
      b1p, W2p, b2p, W3p, b3)


def kernel(sparse_feature, tables, W1, b1, W2, b2, W3, b3):
    tt = jnp.transpose(tables, (0, 2, 1))
    idxT = sparse_feature.astype(jnp.int32).T
    W1p = jnp.pad(jnp.transpose(W1.reshape(H, NF, ED), (1, 2, 0)),
                  ((0, 0), (0, 0), (0, HP - H)))
    b1p = jnp.pad(b1, (0, HP - H))
    b2p = jnp.pad(b2, (0, HP - H))
    W2p = jnp.pad(W2, ((0, HP - H), (0, HP - H)))
    W3p = jnp.pad(W3, ((0, 0), (0, HP - H)))

    gs = []
    fa = 0
    for nf in PHASES:
        P = _tc_project(tt, W1p, fa, nf)
        gs.append(_sc_gather(idxT, P, fa, nf))
        fa += nf
    return _tc_mlp(gs, b1p, W2p, b2p, W3p, b3)
